# baseline (speedup 1.0000x reference)
import functools

import jax
import jax.numpy as jnp
from jax import lax
from jax.experimental import pallas as pl
from jax.experimental.pallas import tpu as pltpu
from jax.experimental.pallas import tpu_sc as plsc

NC = 2
NS = 16
C = 40
CPB = 25
W = 144
ZR = 25
CS = 48


def _rsqrt16(x):
    i = plsc.bitcast(x, jnp.int32)
    y = plsc.bitcast(jnp.int32(0x5F3759DF) - (i >> 1), jnp.float32)
    for _ in range(3):
        y = y * (1.5 - 0.5 * x * y * y)
    return y


def _rsqrt16_fast(x):
    i = plsc.bitcast(x, jnp.int32)
    y = plsc.bitcast(jnp.int32(0x5F3759DF) - (i >> 1), jnp.float32)
    for _ in range(2):
        y = y * (1.5 - 0.5 * x * y * y)
    return y


def _sc_layer_call(fc_tab, nb_tab, row2, col2, w, *, fuse_nb):
    n, d = fc_tab.shape
    e_total = row2.shape[0] * C
    ept = e_total // (NC * NS)
    nch = ept // C
    nblk = nch // CPB
    rpt = n // NS
    assert ept * NC * NS == e_total and nch * C == ept and nblk * CPB == nch
    assert rpt * NS == n and rpt % ZR == 0 and d == 128 and nch % 2 == 0

    mesh = plsc.VectorSubcoreMesh(core_axis_name="c", subcore_axis_name="s")

    scratch = [
        pltpu.VMEM((2 * CPB, C), jnp.int32),
        pltpu.VMEM((2 * CPB, C), jnp.int32),
        pltpu.VMEM((C, d), jnp.bfloat16),
        pltpu.VMEM((C, d), jnp.bfloat16),
        pltpu.VMEM((C, d), jnp.bfloat16),
        pltpu.VMEM((C, d), jnp.bfloat16),
        pltpu.VMEM((C, d), jnp.bfloat16),
        pltpu.VMEM((C, d), jnp.bfloat16),
        pltpu.VMEM((C, W), jnp.float32),
        pltpu.VMEM((C, W), jnp.float32),
        pltpu.VMEM((d,), jnp.float32),
        pltpu.VMEM((ZR, W), jnp.float32),
        pltpu.VMEM((C, d), jnp.bfloat16),
        pltpu.VMEM((CS, 16), jnp.float32),
        pltpu.VMEM((CS, 16), jnp.float32),
        pltpu.VMEM((CS, 16), jnp.float32),
        pltpu.VMEM((CS,), jnp.float32),
        pltpu.VMEM((CS,), jnp.float32),
        pltpu.VMEM_SHARED((n, W), jnp.float32),
        pltpu.SemaphoreType.DMA,
        pltpu.SemaphoreType.DMA,
        pltpu.SemaphoreType.DMA,
        pltpu.SemaphoreType.DMA,
    ]

    @functools.partial(
        pl.kernel,
        out_type=jax.ShapeDtypeStruct((NC, n, W), jnp.float32),
        mesh=mesh,
        scratch_types=scratch,
        compiler_params=pltpu.CompilerParams(use_tc_tiling_on_sc=False,
                                             needs_layout_passes=False),
    )
    def layer(fc_hbm, nb_hbm, row_hbm, col_hbm, w_hbm, out_hbm,
              idr, idc, fcr0, fcr1, fcc0, fcc1, nb0, nb1, out0, out1,
              w_v, z_v, cest, s2b, swb, snb, evb, beb,
              acc_sh, gsem0, gsem1, ssem0, ssem1):
        cid = lax.axis_index("c")
        sid = lax.axis_index("s")
        tid = cid * NS + sid

        fcr = (fcr0, fcr1)
        fcc = (fcc0, fcc1)
        nb = (nb0, nb1)
        out = (out0, out1)
        gsem = (gsem0, gsem1)
        ssem = (ssem0, ssem1)

        zero16 = jnp.zeros((16,), jnp.float32)

        @plsc.parallel_loop(0, ZR * (W // 16))
        def _(i):
            r = i // (W // 16)
            k = i - r * (W // 16)
            z_v[r, pl.ds(k * 16, 16)] = zero16

        r0 = sid * rpt
        for j in range(rpt // ZR):
            pltpu.async_copy(z_v, acc_sh.at[pl.ds(r0 + j * ZR, ZR)], gsem0)
        pltpu.sync_copy(w_hbm, w_v)
        for j in range(rpt // ZR):
            pltpu.make_async_copy(z_v, acc_sh.at[pl.ds(r0 + j * ZR, ZR)],
                                  gsem0).wait()
        plsc.subcore_barrier()

        wv8 = [w_v[pl.ds(j * 16, 16)] for j in range(d // 16)]
        lane = lax.iota(jnp.int32, 16)
        zero16i = jnp.zeros((16,), jnp.int32)
        cbase = tid * nch

        def refill(blk):
            p = lax.rem(blk, 2)
            pltpu.sync_copy(row_hbm.at[pl.ds(cbase + blk * CPB, CPB)],
                            idr.at[pl.ds(p * CPB, CPB)])
            pltpu.sync_copy(col_hbm.at[pl.ds(cbase + blk * CPB, CPB)],
                            idc.at[pl.ds(p * CPB, CPB)])

        def idrow(i):
            return lax.rem(i, 2 * CPB)

        def issue_gather(i, b):
            r = idrow(i)
            cps = [pltpu.async_copy(fc_hbm.at[idr.at[r]], fcr[b], gsem[b]),
                   pltpu.async_copy(fc_hbm.at[idc.at[r]], fcc[b], gsem[b])]
            if not fuse_nb:
                cps.append(
                    pltpu.async_copy(nb_hbm.at[idc.at[r]], nb[b], gsem[b]))

        def wait_gather(i, b):
            r = idrow(i)
            pltpu.make_async_copy(fc_hbm.at[idr.at[r]], fcr[b], gsem[b]).wait()
            pltpu.make_async_copy(fc_hbm.at[idc.at[r]], fcc[b], gsem[b]).wait()
            if not fuse_nb:
                pltpu.make_async_copy(nb_hbm.at[idc.at[r]], nb[b], gsem[b]).wait()

        def compute(i, b):
            @plsc.parallel_loop(0, C, unroll=4)
            def _(e):
                acc2 = [zero16, zero16]
                accw = [zero16, zero16]
                accn = [zero16, zero16]
                for j in range(d // 32):
                    sl = pl.ds(j * 32, 32)
                    a0, a1 = plsc.unpack(fcr[b][e, sl],
                                         format=plsc.PackFormat.INTERLEAVED)
                    b0, b1 = plsc.unpack(fcc[b][e, sl],
                                         format=plsc.PackFormat.INTERLEAVED)
                    if fuse_nb:
                        n0, n1 = b0, b1
                    else:
                        n0, n1 = plsc.unpack(nb[b][e, sl],
                                             format=plsc.PackFormat.INTERLEAVED)
                    ce0 = jnp.abs(a0 + b0)
                    ce1 = jnp.abs(a1 + b1)
                    acc2[0] = acc2[0] + ce0 * ce0
                    acc2[1] = acc2[1] + ce1 * ce1
                    accw[0] = accw[0] + ce0 * wv8[2 * j]
                    accw[1] = accw[1] + ce1 * wv8[2 * j + 1]
                    accn[0] = accn[0] + n0 * ce0
                    accn[1] = accn[1] + n1 * ce1
                    cest[e, sl] = plsc.pack(ce0, ce1,
                                            format=plsc.PackFormat.INTERLEAVED)
                s2b[e, :] = jnp.sum(acc2[0] + acc2[1]) + zero16
                swb[e, :] = jnp.sum(accw[0] + accw[1]) + zero16
                snb[e, :] = jnp.sum(accn[0] + accn[1]) + zero16

            for g in range(C // 16 + 1):
                cnt = min(16, C - g * 16)
                sl = pl.ds(g * 16, 16)
                ridx = g * 16 + lane
                s2v = plsc.load_gather(s2b, [ridx, lane])
                swv = plsc.load_gather(swb, [ridx, lane])
                snv = plsc.load_gather(snb, [ridx, lane])
                inv = _rsqrt16_fast(jnp.maximum(s2v, 1e-24))
                evv = jnp.exp(swv * inv)
                evb[sl] = evv
                beb[sl] = (-2.0 * snv) * evv * inv * inv
                plsc.store_scatter(out[b], [ridx, d + zero16i], evv,
                                   mask=lane < cnt)

            @plsc.parallel_loop(0, C, unroll=4)
            def _(e):
                esplat = e + zero16i
                ev = plsc.load_gather(evb, [esplat])
                beta = plsc.load_gather(beb, [esplat])
                nbsrc = fcc[b] if fuse_nb else nb[b]
                for j in range(d // 32):
                    sl = pl.ds(j * 32, 32)
                    ce0, ce1 = plsc.unpack(cest[e, sl],
                                           format=plsc.PackFormat.INTERLEAVED)
                    n0, n1 = plsc.unpack(nbsrc[e, sl],
                                         format=plsc.PackFormat.INTERLEAVED)
                    out[b][e, pl.ds(j * 32, 16)] = ev * n0 + beta * ce0
                    out[b][e, pl.ds(j * 32 + 16, 16)] = ev * n1 + beta * ce1

        def issue_scatter(i, b):
            pltpu.async_copy(out[b], acc_sh.at[idr.at[idrow(i)]], ssem[b],
                             add=True)

        def wait_scatter(i, b):
            pltpu.make_async_copy(out[b], acc_sh.at[idr.at[idrow(i)]],
                                  ssem[b]).wait()

        refill(0)
        issue_gather(0, 0)

        def pair(it, carry):
            for dd in range(2):
                i = it * 2 + dd
                b = dd
                nxt = i + 1
                wait_gather(i, b)

                @pl.when(jnp.logical_and(lax.rem(nxt, CPB) == 0, nxt < nch))
                def _():
                    refill(nxt // CPB)

                @pl.when(nxt < nch)
                def _():
                    issue_gather(nxt, 1 - b)

                @pl.when(i >= 2)
                def _():
                    wait_scatter(i - 2, b)

                compute(i, b)
                issue_scatter(i, b)
            return carry

        lax.fori_loop(0, nch // 2, pair, 0)
        wait_scatter(nch - 2, 0)
        wait_scatter(nch - 1, 1)
        plsc.subcore_barrier()

        pltpu.sync_copy(acc_sh.at[pl.ds(r0, rpt)],
                        out_hbm.at[cid, pl.ds(r0, rpt)])

    return layer(fc_tab, nb_tab, row2, col2, w)


def _tanh_tc(x):
    n, d = x.shape
    blk = 1000

    def body(x_ref, o_ref):
        o_ref[...] = jnp.tanh(x_ref[...])

    return pl.pallas_call(
        body,
        out_shape=jax.ShapeDtypeStruct((n, d), x.dtype),
        grid=(n // blk,),
        in_specs=[pl.BlockSpec((blk, d), lambda i: (i, 0))],
        out_specs=pl.BlockSpec((blk, d), lambda i: (i, 0)),
    )(x)


def _combine_tc(parts):
    _, n, w = parts.shape
    d = 128
    blk = 1000

    def body(p_ref, o_ref):
        u = p_ref[0, :, :d] + p_ref[1, :, :d]
        s = p_ref[0, :, d] + p_ref[1, :, d]
        s = jnp.where(s == 0.0, 1.0, s)
        o_ref[...] = jnp.tanh(u / s[:, None])

    return pl.pallas_call(
        body,
        out_shape=jax.ShapeDtypeStruct((n, d), jnp.float32),
        grid=(n // blk,),
        in_specs=[pl.BlockSpec((2, blk, w), lambda i: (0, i, 0))],
        out_specs=pl.BlockSpec((blk, d), lambda i: (i, 0)),
    )(parts)


def _to_tab(x):
    n, d = x.shape
    t = x.reshape(n, d // 32, 2, 16).transpose(0, 1, 3, 2).reshape(n, d)
    return t.astype(jnp.bfloat16)


def kernel(features, rel_emb, adj, r_index, r_val, features_c, attn_ent, proxy, gate_w, gate_b):
    e_total = adj.shape[1]
    row2 = adj[0].reshape(e_total // C, C)
    col2 = adj[1].reshape(e_total // C, C)
    feats0 = _tanh_tc(features)
    p0 = _sc_layer_call(_to_tab(features_c), _to_tab(feats0), row2, col2,
                        attn_ent[0, :, 0], fuse_nb=False)
    feats1 = _combine_tc(p0)
    tab1 = _to_tab(feats1)
    p1 = _sc_layer_call(tab1, tab1, row2, col2, attn_ent[1, :, 0],
                        fuse_nb=True)
    feats2 = _combine_tc(p1)
    return jnp.concatenate([feats0, feats1, feats2], axis=-1)


# speedup vs baseline: 1.0795x; 1.0795x over previous
import functools

import jax
import jax.numpy as jnp
from jax import lax
from jax.experimental import pallas as pl
from jax.experimental.pallas import tpu as pltpu
from jax.experimental.pallas import tpu_sc as plsc

NC = 2
NS = 16
C = 40
CPB = 25
W = 144
ZR = 25
CS = 48


def _rsqrt16(x):
    i = plsc.bitcast(x, jnp.int32)
    y = plsc.bitcast(jnp.int32(0x5F3759DF) - (i >> 1), jnp.float32)
    for _ in range(3):
        y = y * (1.5 - 0.5 * x * y * y)
    return y


def _rsqrt16_fast(x):
    i = plsc.bitcast(x, jnp.int32)
    y = plsc.bitcast(jnp.int32(0x5F3759DF) - (i >> 1), jnp.float32)
    for _ in range(2):
        y = y * (1.5 - 0.5 * x * y * y)
    return y


def _sc_layer_call(fc_tab, nb_tab, row2, col2, w, *, fuse_nb):
    n, d = fc_tab.shape
    e_total = row2.shape[0] * C
    ept = e_total // (NC * NS)
    nch = ept // C
    nblk = nch // CPB
    rpt = n // NS
    assert ept * NC * NS == e_total and nch * C == ept and nblk * CPB == nch
    assert rpt * NS == n and rpt % ZR == 0 and d == 128 and nch % 2 == 0

    mesh = plsc.VectorSubcoreMesh(core_axis_name="c", subcore_axis_name="s")

    scratch = [
        pltpu.VMEM((2 * CPB, C), jnp.int32),
        pltpu.VMEM((2 * CPB, C), jnp.int32),
        pltpu.VMEM((C, d), jnp.bfloat16),
        pltpu.VMEM((C, d), jnp.bfloat16),
        pltpu.VMEM((C, d), jnp.bfloat16),
        pltpu.VMEM((C, d), jnp.bfloat16),
        pltpu.VMEM((C, d), jnp.bfloat16),
        pltpu.VMEM((C, d), jnp.bfloat16),
        pltpu.VMEM((C, W), jnp.float32),
        pltpu.VMEM((C, W), jnp.float32),
        pltpu.VMEM((d,), jnp.float32),
        pltpu.VMEM((ZR, W), jnp.float32),
        pltpu.VMEM((C, d), jnp.bfloat16),
        pltpu.VMEM((CS, 16), jnp.float32),
        pltpu.VMEM((CS, 16), jnp.float32),
        pltpu.VMEM((CS, 16), jnp.float32),
        pltpu.VMEM((CS,), jnp.float32),
        pltpu.VMEM((CS,), jnp.float32),
        pltpu.VMEM_SHARED((n, W), jnp.float32),
        pltpu.SemaphoreType.DMA,
        pltpu.SemaphoreType.DMA,
        pltpu.SemaphoreType.DMA,
        pltpu.SemaphoreType.DMA,
    ]

    @functools.partial(
        pl.kernel,
        out_type=jax.ShapeDtypeStruct((NC, n, W), jnp.float32),
        mesh=mesh,
        scratch_types=scratch,
        compiler_params=pltpu.CompilerParams(use_tc_tiling_on_sc=False,
                                             needs_layout_passes=False),
    )
    def layer(fc_hbm, nb_hbm, row_hbm, col_hbm, w_hbm, out_hbm,
              idr, idc, fcr0, fcr1, fcc0, fcc1, nb0, nb1, out0, out1,
              w_v, z_v, cest, s2b, swb, snb, evb, beb,
              acc_sh, gsem0, gsem1, ssem0, ssem1):
        cid = lax.axis_index("c")
        sid = lax.axis_index("s")
        tid = cid * NS + sid

        fcr = (fcr0, fcr1)
        fcc = (fcc0, fcc1)
        nb = (nb0, nb1)
        out = (out0, out1)
        gsem = (gsem0, gsem1)
        ssem = (ssem0, ssem1)

        zero16 = jnp.zeros((16,), jnp.float32)

        @plsc.parallel_loop(0, ZR * (W // 16))
        def _(i):
            r = i // (W // 16)
            k = i - r * (W // 16)
            z_v[r, pl.ds(k * 16, 16)] = zero16

        r0 = sid * rpt
        for j in range(rpt // ZR):
            pltpu.async_copy(z_v, acc_sh.at[pl.ds(r0 + j * ZR, ZR)], gsem0)
        pltpu.sync_copy(w_hbm, w_v)
        for j in range(rpt // ZR):
            pltpu.make_async_copy(z_v, acc_sh.at[pl.ds(r0 + j * ZR, ZR)],
                                  gsem0).wait()
        plsc.subcore_barrier()

        wv8 = [w_v[pl.ds(j * 16, 16)] for j in range(d // 16)]
        lane = lax.iota(jnp.int32, 16)
        zero16i = jnp.zeros((16,), jnp.int32)
        cbase = tid * nch

        def refill(blk):
            p = lax.rem(blk, 2)
            pltpu.sync_copy(row_hbm.at[pl.ds(cbase + blk * CPB, CPB)],
                            idr.at[pl.ds(p * CPB, CPB)])
            pltpu.sync_copy(col_hbm.at[pl.ds(cbase + blk * CPB, CPB)],
                            idc.at[pl.ds(p * CPB, CPB)])

        def idrow(i):
            return lax.rem(i, 2 * CPB)

        def issue_gather(i, b):
            r = idrow(i)
            cps = [pltpu.async_copy(fc_hbm.at[idr.at[r]], fcr[b], gsem[b]),
                   pltpu.async_copy(fc_hbm.at[idc.at[r]], fcc[b], gsem[b])]
            if not fuse_nb:
                cps.append(
                    pltpu.async_copy(nb_hbm.at[idc.at[r]], nb[b], gsem[b]))

        def wait_gather(i, b):
            r = idrow(i)
            pltpu.make_async_copy(fc_hbm.at[idr.at[r]], fcr[b], gsem[b]).wait()
            pltpu.make_async_copy(fc_hbm.at[idc.at[r]], fcc[b], gsem[b]).wait()
            if not fuse_nb:
                pltpu.make_async_copy(nb_hbm.at[idc.at[r]], nb[b], gsem[b]).wait()

        def compute(i, b):
            @plsc.parallel_loop(0, C, unroll=2)
            def _(e):
                acc2 = [zero16, zero16]
                accw = [zero16, zero16]
                accn = [zero16, zero16]
                for j in range(d // 32):
                    sl = pl.ds(j * 32, 32)
                    a0, a1 = plsc.unpack(fcr[b][e, sl],
                                         format=plsc.PackFormat.INTERLEAVED)
                    b0, b1 = plsc.unpack(fcc[b][e, sl],
                                         format=plsc.PackFormat.INTERLEAVED)
                    if fuse_nb:
                        n0, n1 = b0, b1
                    else:
                        n0, n1 = plsc.unpack(nb[b][e, sl],
                                             format=plsc.PackFormat.INTERLEAVED)
                    ce0 = jnp.abs(a0 + b0)
                    ce1 = jnp.abs(a1 + b1)
                    acc2[0] = acc2[0] + ce0 * ce0
                    acc2[1] = acc2[1] + ce1 * ce1
                    accw[0] = accw[0] + ce0 * wv8[2 * j]
                    accw[1] = accw[1] + ce1 * wv8[2 * j + 1]
                    accn[0] = accn[0] + n0 * ce0
                    accn[1] = accn[1] + n1 * ce1
                    cest[e, sl] = plsc.pack(ce0, ce1,
                                            format=plsc.PackFormat.INTERLEAVED)
                s2b[e, :] = jnp.sum(acc2[0] + acc2[1]) + zero16
                swb[e, :] = jnp.sum(accw[0] + accw[1]) + zero16
                snb[e, :] = jnp.sum(accn[0] + accn[1]) + zero16

            for g in range(C // 16 + 1):
                cnt = min(16, C - g * 16)
                sl = pl.ds(g * 16, 16)
                ridx = g * 16 + lane
                s2v = plsc.load_gather(s2b, [ridx, lane])
                swv = plsc.load_gather(swb, [ridx, lane])
                snv = plsc.load_gather(snb, [ridx, lane])
                inv = _rsqrt16_fast(jnp.maximum(s2v, 1e-24))
                evv = jnp.exp(swv * inv)
                evb[sl] = evv
                beb[sl] = (-2.0 * snv) * evv * inv * inv
                plsc.store_scatter(out[b], [ridx, d + zero16i], evv,
                                   mask=lane < cnt)

            @plsc.parallel_loop(0, C, unroll=2)
            def _(e):
                esplat = e + zero16i
                ev = plsc.load_gather(evb, [esplat])
                beta = plsc.load_gather(beb, [esplat])
                nbsrc = fcc[b] if fuse_nb else nb[b]
                for j in range(d // 32):
                    sl = pl.ds(j * 32, 32)
                    ce0, ce1 = plsc.unpack(cest[e, sl],
                                           format=plsc.PackFormat.INTERLEAVED)
                    n0, n1 = plsc.unpack(nbsrc[e, sl],
                                         format=plsc.PackFormat.INTERLEAVED)
                    out[b][e, pl.ds(j * 32, 16)] = ev * n0 + beta * ce0
                    out[b][e, pl.ds(j * 32 + 16, 16)] = ev * n1 + beta * ce1

        def issue_scatter(i, b):
            pltpu.async_copy(out[b], acc_sh.at[idr.at[idrow(i)]], ssem[b],
                             add=True)

        def wait_scatter(i, b):
            pltpu.make_async_copy(out[b], acc_sh.at[idr.at[idrow(i)]],
                                  ssem[b]).wait()

        refill(0)
        issue_gather(0, 0)

        def pair(it, carry):
            for dd in range(2):
                i = it * 2 + dd
                b = dd
                nxt = i + 1
                wait_gather(i, b)

                @pl.when(jnp.logical_and(lax.rem(nxt, CPB) == 0, nxt < nch))
                def _():
                    refill(nxt // CPB)

                @pl.when(nxt < nch)
                def _():
                    issue_gather(nxt, 1 - b)

                @pl.when(i >= 2)
                def _():
                    wait_scatter(i - 2, b)

                compute(i, b)
                issue_scatter(i, b)
            return carry

        lax.fori_loop(0, nch // 2, pair, 0)
        wait_scatter(nch - 2, 0)
        wait_scatter(nch - 1, 1)
        plsc.subcore_barrier()

        pltpu.sync_copy(acc_sh.at[pl.ds(r0, rpt)],
                        out_hbm.at[cid, pl.ds(r0, rpt)])

    return layer(fc_tab, nb_tab, row2, col2, w)


def _tanh_tc(x):
    n, d = x.shape
    blk = 1000

    def body(x_ref, o_ref):
        o_ref[...] = jnp.tanh(x_ref[...])

    return pl.pallas_call(
        body,
        out_shape=jax.ShapeDtypeStruct((n, d), x.dtype),
        grid=(n // blk,),
        in_specs=[pl.BlockSpec((blk, d), lambda i: (i, 0))],
        out_specs=pl.BlockSpec((blk, d), lambda i: (i, 0)),
    )(x)


def _combine_tc(parts):
    _, n, w = parts.shape
    d = 128
    blk = 1000

    def body(p_ref, o_ref):
        u = p_ref[0, :, :d] + p_ref[1, :, :d]
        s = p_ref[0, :, d] + p_ref[1, :, d]
        s = jnp.where(s == 0.0, 1.0, s)
        o_ref[...] = jnp.tanh(u / s[:, None])

    return pl.pallas_call(
        body,
        out_shape=jax.ShapeDtypeStruct((n, d), jnp.float32),
        grid=(n // blk,),
        in_specs=[pl.BlockSpec((2, blk, w), lambda i: (0, i, 0))],
        out_specs=pl.BlockSpec((blk, d), lambda i: (i, 0)),
    )(parts)


def _to_tab(x):
    n, d = x.shape
    t = x.reshape(n, d // 32, 2, 16).transpose(0, 1, 3, 2).reshape(n, d)
    return t.astype(jnp.bfloat16)


def kernel(features, rel_emb, adj, r_index, r_val, features_c, attn_ent, proxy, gate_w, gate_b):
    e_total = adj.shape[1]
    row2 = adj[0].reshape(e_total // C, C)
    col2 = adj[1].reshape(e_total // C, C)
    feats0 = _tanh_tc(features)
    p0 = _sc_layer_call(_to_tab(features_c), _to_tab(feats0), row2, col2,
                        attn_ent[0, :, 0], fuse_nb=False)
    feats1 = _combine_tc(p0)
    tab1 = _to_tab(feats1)
    p1 = _sc_layer_call(tab1, tab1, row2, col2, attn_ent[1, :, 0],
                        fuse_nb=True)
    feats2 = _combine_tc(p1)
    return jnp.concatenate([feats0, feats1, feats2], axis=-1)
